# baseline (speedup 1.0000x reference)
import functools

import jax
import jax.numpy as jnp
from jax import lax
from jax.experimental import pallas as pl
from jax.experimental.pallas import tpu as pltpu
from jax.experimental.pallas import tpu_sc as plsc

N = 10000
E = 160000
C = 128
A = 10
KC = 4
CH = C // KC
AVG = 16.0

NC = 2
NS = 16
EP = E // NS
BE = 80
NB = EP // BE
NP = 10240
NPT = NP // NS


def _xup_body(nf_ref, wup_ref, out_ref):
    out_ref[...] = jnp.dot(nf_ref[...], wup_ref[0],
                           preferred_element_type=jnp.float32)


def _x_chunks(node_feats, W_up):
    BN = 1000
    return pl.pallas_call(
        _xup_body,
        grid=(KC, N // BN),
        in_specs=[
            pl.BlockSpec((BN, C), lambda k, i: (i, 0)),
            pl.BlockSpec((1, C, CH), lambda k, i: (k, 0, 0)),
        ],
        out_specs=pl.BlockSpec((BN, CH), lambda k, i: (k * (N // BN) + i, 0)),
        out_shape=jax.ShapeDtypeStruct((KC * N, CH), jnp.float32),
    )(node_feats, W_up)


def _mlp_body(efa_ref, efb_ref, sa_ref, sb_ref, w1_ref, b1_ref, w2_ref,
              b2_ref, w3_ref, b3_ref, w4_ref, b4_ref, out_ref):
    def trunk(ef):
        h = jax.nn.silu(jnp.dot(ef, w1_ref[...],
                                preferred_element_type=jnp.float32)
                        + b1_ref[...])
        h = jax.nn.silu(jnp.dot(h, w2_ref[...],
                                preferred_element_type=jnp.float32)
                        + b2_ref[...])
        return jax.nn.silu(jnp.dot(h, w3_ref[...],
                                   preferred_element_type=jnp.float32)
                           + b3_ref[...])

    ha = trunk(efa_ref[...])
    hb = trunk(efb_ref[...])
    col = lax.broadcasted_iota(jnp.int32, (ha.shape[0], 2 * CH), 1)
    sca = jnp.where(col < CH, sa_ref[...][:, 0:1], 1.0)
    scb = jnp.where(col < CH, sb_ref[...][:, 0:1], 1.0)
    for k in range(KC):
        wa = (jnp.dot(ha, w4_ref[k], preferred_element_type=jnp.float32)
              + b4_ref[k]) * sca
        wb = (jnp.dot(hb, w4_ref[k], preferred_element_type=jnp.float32)
              + b4_ref[k]) * scb
        out_ref[k] = jnp.concatenate([wa, wb], axis=1)


def _edge_weights(edge_feats, edge_attrs, W1, b1, W2, b2, W3, b3, W4r, b4r):
    E2 = E // 2
    BEm = 1000
    nb = E2 // BEm
    return pl.pallas_call(
        _mlp_body,
        grid=(nb,),
        in_specs=[
            pl.BlockSpec((BEm, 8), lambda i: (i, 0)),
            pl.BlockSpec((BEm, 8), lambda i, _nb=nb: (_nb + i, 0)),
            pl.BlockSpec((BEm, 4), lambda i: (i, 0)),
            pl.BlockSpec((BEm, 4), lambda i, _nb=nb: (_nb + i, 0)),
            pl.BlockSpec((8, 64), lambda i: (0, 0)),
            pl.BlockSpec((1, 64), lambda i: (0, 0)),
            pl.BlockSpec((64, 64), lambda i: (0, 0)),
            pl.BlockSpec((1, 64), lambda i: (0, 0)),
            pl.BlockSpec((64, 64), lambda i: (0, 0)),
            pl.BlockSpec((1, 64), lambda i: (0, 0)),
            pl.BlockSpec((KC, 64, 2 * CH), lambda i: (0, 0, 0)),
            pl.BlockSpec((KC, 1, 2 * CH), lambda i: (0, 0, 0)),
        ],
        out_specs=pl.BlockSpec((KC, BEm, 4 * CH), lambda i: (0, i, 0)),
        out_shape=jax.ShapeDtypeStruct((KC, E2, 4 * CH), jnp.float32),
    )(edge_feats, edge_feats, edge_attrs, edge_attrs,
      W1, b1, W2, b2, W3, b3, W4r, b4r)


def _sc_body(xflat, wflat, sh3, send, recv, out,
             isv0, isv1, rcv0, rcv1, xj0, xj1, w0_, w1_, sh0_, sh1_,
             m0_, m1_, z_v, acc_sh,
             is0, is1, ls0, ls1, gs0, gs1, ss0, ss1):
    c = lax.axis_index("c")
    s = lax.axis_index("s")
    isv = (isv0, isv1)
    rcv = (rcv0, rcv1)
    xj = (xj0, xj1)
    w_v = (w0_, w1_)
    sh_v = (sh0_, sh1_)
    m_v = (m0_, m1_)
    is_ = (is0, is1)
    ls = (ls0, ls1)
    gs = (gs0, gs1)
    ss = (ss0, ss1)

    def _zb(i, _):
        r = i // (C // 16)
        g = i % (C // 16)
        z_v[r, pl.ds(g * 16, 16)] = jnp.zeros((16,), jnp.float32)
        return 0
    lax.fori_loop(0, 32 * (C // 16), _zb, 0)

    for kk in range(KC // NC):
        chunk = NC * c + kk
        for t in range(NPT // 32):
            pltpu.sync_copy(z_v, acc_sh.at[pl.ds(s * NPT + t * 32, 32)])
        plsc.subcore_barrier()

        xoff = chunk * N
        woff = chunk * (E // 2)
        ebase = s * EP

        H = BE // 2

        def is_issue(b, q):
            pltpu.async_copy(send.at[pl.ds(ebase + b * BE, BE)],
                             isv[q], is_[q])

        def is_wait(q):
            pltpu.make_async_copy(send.at[pl.ds(0, BE)], isv[q],
                                  is_[q]).wait()

        def lin_issue(b, q):
            base = ebase + b * BE
            pltpu.async_copy(recv.at[pl.ds(base, BE)], rcv[q], ls[q])
            pltpu.async_copy(sh3.at[pl.ds(3 * base, 3 * BE)],
                             sh_v[q].at[pl.ds(0, 3 * BE)], ls[q])
            pltpu.async_copy(wflat.at[pl.ds(woff + s * (EP // 2) + b * H, H)],
                             w_v[q], ls[q])

        def lin_wait(q):
            pltpu.make_async_copy(recv.at[pl.ds(0, BE)], rcv[q],
                                  ls[q]).wait()
            pltpu.make_async_copy(sh3.at[pl.ds(0, 3 * BE)],
                                  sh_v[q].at[pl.ds(0, 3 * BE)], ls[q]).wait()
            pltpu.make_async_copy(wflat.at[pl.ds(0, H)], w_v[q],
                                  ls[q]).wait()

        def gather_issue(q):
            for gg in range(BE // 16):
                isv[q][pl.ds(gg * 16, 16)] = (
                    isv[q][pl.ds(gg * 16, 16)] + xoff)
            pltpu.async_copy(xflat.at[isv[q]], xj[q], gs[q])

        def gather_wait(q):
            pltpu.make_async_copy(xflat.at[isv[q]], xj[q], gs[q]).wait()

        def scatter_issue(q):
            pltpu.async_copy(m_v[q], acc_sh.at[rcv[q]], ss[q], add=True)

        def scatter_wait(q):
            pltpu.make_async_copy(m_v[q], acc_sh.at[rcv[q]], ss[q]).wait()

        def compute(q):
            xjq, wq, shq, mq = xj[q], w_v[q], sh_v[q], m_v[q]

            def _pair(ep, _):
                for half in range(2):
                    e = ep + half * (BE // 2)
                    coff = half * 2 * CH
                    shx = shq[pl.ds(e, 16)][0]
                    shy = shq[pl.ds(e + BE, 16)][0]
                    shz = shq[pl.ds(e + 2 * BE, 16)][0]
                    for j in range(CH // 16):
                        xv = xjq[e, pl.ds(j * 16, 16)]
                        w0v = wq[ep, pl.ds(coff + j * 16, 16)]
                        w1v = wq[ep, pl.ds(coff + CH + j * 16, 16)]
                        mq[e, pl.ds(j * 16, 16)] = w0v * xv
                        t1 = w1v * xv
                        mq[e, pl.ds(CH + j * 16, 16)] = t1 * shx
                        mq[e, pl.ds(2 * CH + j * 16, 16)] = t1 * shy
                        mq[e, pl.ds(3 * CH + j * 16, 16)] = t1 * shz
                return 0
            lax.fori_loop(0, BE // 2, _pair, 0)

        is_issue(0, 0)
        is_issue(1, 1)
        lin_issue(0, 0)
        is_wait(0)
        gather_issue(0)

        def pair(t, _):
            for p in (0, 1):
                g = 2 * t + p
                lin_wait(p)
                gather_wait(p)
                compute(p)
                scatter_issue(p)
                if p == 0:
                    @pl.when(g > 0)
                    def _():
                        scatter_wait(1)
                else:
                    scatter_wait(0)
                is_wait(1 - p)
                gather_issue(1 - p)
                lin_issue(g + 1, 1 - p)
                if p == 0:
                    is_issue(g + 2, p)
                else:
                    @pl.when(g < NB - 2)
                    def _():
                        is_issue(g + 2, p)
            return 0
        lax.fori_loop(0, NB // 2, pair, 0)
        lin_wait(0)
        gather_wait(0)
        compute(0)
        scatter_issue(0)
        scatter_wait(1)
        scatter_wait(0)

        plsc.subcore_barrier()
        pltpu.sync_copy(acc_sh.at[pl.ds(s * NPT, NPT)],
                        out.at[pl.ds(chunk * NP + s * NPT, NPT)])


def _sc_scatter(xflat, wflat, sh3, send, recv):
    mesh = plsc.VectorSubcoreMesh(core_axis_name="c", subcore_axis_name="s")
    run = functools.partial(
        pl.kernel,
        out_type=jax.ShapeDtypeStruct((KC * NP, 4 * CH), jnp.float32),
        mesh=mesh,
        scratch_types=(
            [pltpu.VMEM((BE,), jnp.int32)] * 4
            + [pltpu.VMEM((BE, CH), jnp.float32)] * 2
            + [pltpu.VMEM((BE // 2, 4 * CH), jnp.float32)] * 2
            + [pltpu.VMEM((3 * BE + 16,), jnp.float32)] * 2
            + [pltpu.VMEM((BE, 4 * CH), jnp.float32)] * 2
            + [pltpu.VMEM((32, 4 * CH), jnp.float32)]
            + [pltpu.VMEM_SHARED((NP, 4 * CH), jnp.float32)]
            + [pltpu.SemaphoreType.DMA] * 8
        ),
        compiler_params=pltpu.CompilerParams(use_tc_tiling_on_sc=False),
    )(_sc_body)
    return run(xflat, wflat, sh3, send, recv)


def _post_body(msg_ref, attrs_ref, wl0_ref, wl1_ref, ws0_ref, ws1_ref, out_ref):
    attrs = attrs_ref[...]

    def skip(m, ws_ref):
        t = jnp.concatenate(
            [(m * attrs[:, v:v + 1]).astype(jnp.bfloat16) for v in range(A)],
            axis=1)
        return jnp.dot(t, ws_ref[...], preferred_element_type=jnp.float32)

    for i in range(4):
        m = jnp.concatenate(
            [msg_ref[k][:, i * CH:(i + 1) * CH] for k in range(KC)], axis=1)
        wl = wl0_ref if i == 0 else wl1_ref
        m = jnp.dot(m, wl[...], preferred_element_type=jnp.float32)
        out_ref[i] = skip(m, ws0_ref if i == 0 else ws1_ref)


def _post(msgc, node_attrs, WL0s, WL1s, WS0r, WS1r):
    BN = 400
    return pl.pallas_call(
        _post_body,
        grid=(N // BN,),
        in_specs=[
            pl.BlockSpec((KC, BN, 4 * CH), lambda i: (0, i, 0)),
            pl.BlockSpec((BN, A), lambda i: (i, 0)),
            pl.BlockSpec((C, C), lambda i: (0, 0)),
            pl.BlockSpec((C, C), lambda i: (0, 0)),
            pl.BlockSpec((A * C, C), lambda i: (0, 0)),
            pl.BlockSpec((A * C, C), lambda i: (0, 0)),
        ],
        out_specs=pl.BlockSpec((4, BN, C), lambda i: (0, i, 0)),
        out_shape=jax.ShapeDtypeStruct((4, N, C), jnp.float32),
    )(msgc, node_attrs, WL0s, WL1s, WS0r, WS1r)


def kernel(node_attrs, node_feats, edge_attrs, edge_feats, edge_index,
           W_up, W1, b1, W2, b2, W3, b3, W4, b4, WL0, WL1, WS0, WS1):
    cols = jnp.concatenate(
        [jnp.concatenate([jnp.arange(k * CH, (k + 1) * CH),
                          jnp.arange(C + k * CH, C + (k + 1) * CH)])
         for k in range(KC)])
    W4r = W4[:, cols].reshape(64, KC, 2 * CH).transpose(1, 0, 2)
    b4r = b4[cols].reshape(KC, 1, 2 * CH)
    WL0s = WL0 / AVG
    WL1s = WL1 / AVG
    WS0r = jnp.transpose(WS0, (1, 0, 2)).reshape(A * C, C).astype(jnp.bfloat16)
    WS1r = jnp.transpose(WS1, (1, 0, 2)).reshape(A * C, C).astype(jnp.bfloat16)

    E2 = E // 2
    H = BE // 2
    NBK = E2 // H
    sender = jnp.transpose(
        edge_index[0].astype(jnp.int32).reshape(2, NBK, H),
        (1, 0, 2)).reshape(E)
    recv = jnp.transpose(
        edge_index[1].astype(jnp.int32).reshape(2, NBK, H),
        (1, 0, 2)).reshape(E)
    sh3 = jnp.transpose(edge_attrs[:, 1:4].reshape(2, NBK, H, 3),
                        (1, 3, 0, 2)).reshape(3 * E)

    W_upr = W_up.reshape(C, KC, CH).transpose(1, 0, 2)
    xflat = _x_chunks(node_feats, W_upr)
    wflat = _edge_weights(edge_feats, edge_attrs, W1, b1[None, :],
                          W2, b2[None, :], W3, b3[None, :],
                          W4r, b4r).reshape(KC * E2, 4 * CH)
    msgc = _sc_scatter(xflat, wflat, sh3, sender, recv)
    out4 = _post(msgc.reshape(KC, NP, 4 * CH), node_attrs,
                 WL0s, WL1s, WS0r, WS1r)
    return jnp.transpose(out4, (1, 2, 0))


# speedup vs baseline: 1.0642x; 1.0642x over previous
import functools

import jax
import jax.numpy as jnp
from jax import lax
from jax.experimental import pallas as pl
from jax.experimental.pallas import tpu as pltpu
from jax.experimental.pallas import tpu_sc as plsc

N = 10000
E = 160000
C = 128
A = 10
KC = 4
CH = C // KC
AVG = 16.0

NC = 2
NS = 16
EP = E // NS
BE = 80
NB = EP // BE
NP = 10240
NPT = NP // NS


def _xup_body(nf_ref, wup_ref, out_ref):
    out_ref[...] = jnp.dot(nf_ref[...], wup_ref[0],
                           preferred_element_type=jnp.float32)


def _x_chunks(node_feats, W_up):
    BN = 1000
    return pl.pallas_call(
        _xup_body,
        grid=(KC, N // BN),
        in_specs=[
            pl.BlockSpec((BN, C), lambda k, i: (i, 0)),
            pl.BlockSpec((1, C, CH), lambda k, i: (k, 0, 0)),
        ],
        out_specs=pl.BlockSpec((BN, CH), lambda k, i: (k * (N // BN) + i, 0)),
        out_shape=jax.ShapeDtypeStruct((KC * N, CH), jnp.float32),
    )(node_feats, W_up)


def _mlp_body(efa_ref, efb_ref, sa_ref, sb_ref, w1_ref, b1_ref, w2_ref,
              b2_ref, w3_ref, b3_ref, w4_ref, b4_ref, out_ref):
    ef = jnp.concatenate([efa_ref[...], efb_ref[...]], axis=0)
    h = jax.nn.silu(jnp.dot(ef, w1_ref[...],
                            preferred_element_type=jnp.float32) + b1_ref[...])
    h = jax.nn.silu(jnp.dot(h, w2_ref[...],
                            preferred_element_type=jnp.float32) + b2_ref[...])
    h = jax.nn.silu(jnp.dot(h, w3_ref[...],
                            preferred_element_type=jnp.float32) + b3_ref[...])
    w = jnp.dot(h, w4_ref[...], preferred_element_type=jnp.float32) \
        + b4_ref[...]
    s0 = jnp.concatenate([sa_ref[...], sb_ref[...]], axis=0)[:, 0:1]
    col = lax.broadcasted_iota(jnp.int32, w.shape, 1)
    w = w * jnp.where((col % (2 * CH)) < CH, s0, 1.0)
    half = w.shape[0] // 2
    for k in range(KC):
        out_ref[k] = jnp.concatenate(
            [w[:half, k * 2 * CH:(k + 1) * 2 * CH],
             w[half:, k * 2 * CH:(k + 1) * 2 * CH]], axis=1)


def _edge_weights(edge_feats, edge_attrs, W1, b1, W2, b2, W3, b3, W4f, b4f):
    E2 = E // 2
    BEm = 1000
    nb = E2 // BEm
    return pl.pallas_call(
        _mlp_body,
        grid=(nb,),
        in_specs=[
            pl.BlockSpec((BEm, 8), lambda i: (i, 0)),
            pl.BlockSpec((BEm, 8), lambda i, _nb=nb: (_nb + i, 0)),
            pl.BlockSpec((BEm, 4), lambda i: (i, 0)),
            pl.BlockSpec((BEm, 4), lambda i, _nb=nb: (_nb + i, 0)),
            pl.BlockSpec((8, 64), lambda i: (0, 0)),
            pl.BlockSpec((1, 64), lambda i: (0, 0)),
            pl.BlockSpec((64, 64), lambda i: (0, 0)),
            pl.BlockSpec((1, 64), lambda i: (0, 0)),
            pl.BlockSpec((64, 64), lambda i: (0, 0)),
            pl.BlockSpec((1, 64), lambda i: (0, 0)),
            pl.BlockSpec((64, 2 * C), lambda i: (0, 0)),
            pl.BlockSpec((1, 2 * C), lambda i: (0, 0)),
        ],
        out_specs=pl.BlockSpec((KC, BEm, 4 * CH), lambda i: (0, i, 0)),
        out_shape=jax.ShapeDtypeStruct((KC, E2, 4 * CH), jnp.float32),
    )(edge_feats, edge_feats, edge_attrs, edge_attrs,
      W1, b1, W2, b2, W3, b3, W4f, b4f)


def _sc_body(xflat, wflat, sh3, send, recv, out,
             isv0, isv1, rcv0, rcv1, xj0, xj1, w0_, w1_, sh0_, sh1_,
             m0_, m1_, z_v, acc_sh,
             is0, is1, ls0, ls1, gs0, gs1, ss0, ss1):
    c = lax.axis_index("c")
    s = lax.axis_index("s")
    isv = (isv0, isv1)
    rcv = (rcv0, rcv1)
    xj = (xj0, xj1)
    w_v = (w0_, w1_)
    sh_v = (sh0_, sh1_)
    m_v = (m0_, m1_)
    is_ = (is0, is1)
    ls = (ls0, ls1)
    gs = (gs0, gs1)
    ss = (ss0, ss1)

    def _zb(i, _):
        r = i // (C // 16)
        g = i % (C // 16)
        z_v[r, pl.ds(g * 16, 16)] = jnp.zeros((16,), jnp.float32)
        return 0
    lax.fori_loop(0, 32 * (C // 16), _zb, 0)

    for kk in range(KC // NC):
        chunk = NC * c + kk
        for t in range(NPT // 32):
            pltpu.sync_copy(z_v, acc_sh.at[pl.ds(s * NPT + t * 32, 32)])
        plsc.subcore_barrier()

        xoff = chunk * N
        woff = chunk * (E // 2)
        ebase = s * EP

        H = BE // 2

        def is_issue(b, q):
            bh = s * (EP // 2) + b * H
            pltpu.async_copy(send.at[pl.ds(bh, H)],
                             isv[q].at[pl.ds(0, H)], is_[q])
            pltpu.async_copy(send.at[pl.ds(E // 2 + bh, H)],
                             isv[q].at[pl.ds(H, H)], is_[q])

        def is_wait(q):
            pltpu.make_async_copy(send.at[pl.ds(0, H)],
                                  isv[q].at[pl.ds(0, H)], is_[q]).wait()
            pltpu.make_async_copy(send.at[pl.ds(0, H)],
                                  isv[q].at[pl.ds(H, H)], is_[q]).wait()

        def lin_issue(b, q):
            bh = s * (EP // 2) + b * H
            pltpu.async_copy(recv.at[pl.ds(bh, H)],
                             rcv[q].at[pl.ds(0, H)], ls[q])
            pltpu.async_copy(recv.at[pl.ds(E // 2 + bh, H)],
                             rcv[q].at[pl.ds(H, H)], ls[q])
            pltpu.async_copy(sh3.at[pl.ds(3 * bh, 3 * H)],
                             sh_v[q].at[pl.ds(0, 3 * H)], ls[q])
            pltpu.async_copy(sh3.at[pl.ds(3 * (E // 2) + 3 * bh, 3 * H)],
                             sh_v[q].at[pl.ds(3 * H, 3 * H)], ls[q])
            pltpu.async_copy(wflat.at[pl.ds(woff + bh, H)], w_v[q], ls[q])

        def lin_wait(q):
            pltpu.make_async_copy(recv.at[pl.ds(0, H)],
                                  rcv[q].at[pl.ds(0, H)], ls[q]).wait()
            pltpu.make_async_copy(recv.at[pl.ds(0, H)],
                                  rcv[q].at[pl.ds(H, H)], ls[q]).wait()
            pltpu.make_async_copy(sh3.at[pl.ds(0, 3 * H)],
                                  sh_v[q].at[pl.ds(0, 3 * H)], ls[q]).wait()
            pltpu.make_async_copy(sh3.at[pl.ds(0, 3 * H)],
                                  sh_v[q].at[pl.ds(3 * H, 3 * H)],
                                  ls[q]).wait()
            pltpu.make_async_copy(wflat.at[pl.ds(0, H)], w_v[q],
                                  ls[q]).wait()

        def gather_issue(q):
            for gg in range(BE // 16):
                isv[q][pl.ds(gg * 16, 16)] = (
                    isv[q][pl.ds(gg * 16, 16)] + xoff)
            pltpu.async_copy(xflat.at[isv[q]], xj[q], gs[q])

        def gather_wait(q):
            pltpu.make_async_copy(xflat.at[isv[q]], xj[q], gs[q]).wait()

        def scatter_issue(q):
            pltpu.async_copy(m_v[q], acc_sh.at[rcv[q]], ss[q], add=True)

        def scatter_wait(q):
            pltpu.make_async_copy(m_v[q], acc_sh.at[rcv[q]], ss[q]).wait()

        def compute(q):
            xjq, wq, shq, mq = xj[q], w_v[q], sh_v[q], m_v[q]

            def _pair(ep, _):
                for half in range(2):
                    e = ep + half * (BE // 2)
                    coff = half * 2 * CH
                    soff = half * BE
                    shx = shq[pl.ds(e + soff, 16)][0]
                    shy = shq[pl.ds(e + soff + BE // 2, 16)][0]
                    shz = shq[pl.ds(e + soff + BE, 16)][0]
                    for j in range(CH // 16):
                        xv = xjq[e, pl.ds(j * 16, 16)]
                        w0v = wq[ep, pl.ds(coff + j * 16, 16)]
                        w1v = wq[ep, pl.ds(coff + CH + j * 16, 16)]
                        mq[e, pl.ds(j * 16, 16)] = w0v * xv
                        t1 = w1v * xv
                        mq[e, pl.ds(CH + j * 16, 16)] = t1 * shx
                        mq[e, pl.ds(2 * CH + j * 16, 16)] = t1 * shy
                        mq[e, pl.ds(3 * CH + j * 16, 16)] = t1 * shz
                return 0
            lax.fori_loop(0, BE // 2, _pair, 0)

        is_issue(0, 0)
        is_issue(1, 1)
        lin_issue(0, 0)
        is_wait(0)
        gather_issue(0)

        def pair(t, _):
            for p in (0, 1):
                g = 2 * t + p
                lin_wait(p)
                gather_wait(p)
                compute(p)
                scatter_issue(p)
                if p == 0:
                    @pl.when(g > 0)
                    def _():
                        scatter_wait(1)
                else:
                    scatter_wait(0)
                is_wait(1 - p)
                gather_issue(1 - p)
                lin_issue(g + 1, 1 - p)
                if p == 0:
                    is_issue(g + 2, p)
                else:
                    @pl.when(g < NB - 2)
                    def _():
                        is_issue(g + 2, p)
            return 0
        lax.fori_loop(0, NB // 2, pair, 0)
        lin_wait(0)
        gather_wait(0)
        compute(0)
        scatter_issue(0)
        scatter_wait(1)
        scatter_wait(0)

        plsc.subcore_barrier()
        pltpu.sync_copy(acc_sh.at[pl.ds(s * NPT, NPT)],
                        out.at[pl.ds(chunk * NP + s * NPT, NPT)])


def _sc_scatter(xflat, wflat, sh3, send, recv):
    mesh = plsc.VectorSubcoreMesh(core_axis_name="c", subcore_axis_name="s")
    run = functools.partial(
        pl.kernel,
        out_type=jax.ShapeDtypeStruct((KC * NP, 4 * CH), jnp.float32),
        mesh=mesh,
        scratch_types=(
            [pltpu.VMEM((BE,), jnp.int32)] * 4
            + [pltpu.VMEM((BE, CH), jnp.float32)] * 2
            + [pltpu.VMEM((BE // 2, 4 * CH), jnp.float32)] * 2
            + [pltpu.VMEM((3 * BE + 16,), jnp.float32)] * 2
            + [pltpu.VMEM((BE, 4 * CH), jnp.float32)] * 2
            + [pltpu.VMEM((32, 4 * CH), jnp.float32)]
            + [pltpu.VMEM_SHARED((NP, 4 * CH), jnp.float32)]
            + [pltpu.SemaphoreType.DMA] * 8
        ),
        compiler_params=pltpu.CompilerParams(use_tc_tiling_on_sc=False),
    )(_sc_body)
    return run(xflat, wflat, sh3, send, recv)


def _post_body(msg_ref, attrs_ref, wl0_ref, wl1_ref, ws0_ref, ws1_ref, out_ref):
    attrs = attrs_ref[...]

    def skip(m, ws_ref):
        t = jnp.concatenate(
            [(m * attrs[:, v:v + 1]).astype(jnp.bfloat16) for v in range(A)],
            axis=1)
        return jnp.dot(t, ws_ref[...], preferred_element_type=jnp.float32)

    for i in range(4):
        m = jnp.concatenate(
            [msg_ref[k][:, i * CH:(i + 1) * CH] for k in range(KC)], axis=1)
        wl = wl0_ref if i == 0 else wl1_ref
        m = jnp.dot(m, wl[...], preferred_element_type=jnp.float32)
        out_ref[i] = skip(m, ws0_ref if i == 0 else ws1_ref)


def _post(msgc, node_attrs, WL0s, WL1s, WS0r, WS1r):
    BN = 400
    return pl.pallas_call(
        _post_body,
        grid=(N // BN,),
        in_specs=[
            pl.BlockSpec((KC, BN, 4 * CH), lambda i: (0, i, 0)),
            pl.BlockSpec((BN, A), lambda i: (i, 0)),
            pl.BlockSpec((C, C), lambda i: (0, 0)),
            pl.BlockSpec((C, C), lambda i: (0, 0)),
            pl.BlockSpec((A * C, C), lambda i: (0, 0)),
            pl.BlockSpec((A * C, C), lambda i: (0, 0)),
        ],
        out_specs=pl.BlockSpec((4, BN, C), lambda i: (0, i, 0)),
        out_shape=jax.ShapeDtypeStruct((4, N, C), jnp.float32),
    )(msgc, node_attrs, WL0s, WL1s, WS0r, WS1r)


def kernel(node_attrs, node_feats, edge_attrs, edge_feats, edge_index,
           W_up, W1, b1, W2, b2, W3, b3, W4, b4, WL0, WL1, WS0, WS1):
    cols = jnp.concatenate(
        [jnp.concatenate([jnp.arange(k * CH, (k + 1) * CH),
                          jnp.arange(C + k * CH, C + (k + 1) * CH)])
         for k in range(KC)])
    W4f = W4[:, cols]
    b4f = b4[cols][None, :]
    WL0s = WL0 / AVG
    WL1s = WL1 / AVG
    WS0r = jnp.transpose(WS0, (1, 0, 2)).reshape(A * C, C).astype(jnp.bfloat16)
    WS1r = jnp.transpose(WS1, (1, 0, 2)).reshape(A * C, C).astype(jnp.bfloat16)

    E2 = E // 2
    sender = edge_index[0].astype(jnp.int32)
    recv = edge_index[1].astype(jnp.int32)
    H = BE // 2
    sh3 = jnp.transpose(edge_attrs[:, 1:4].reshape(E // H, H, 3),
                        (0, 2, 1)).reshape(3 * E)

    W_upr = W_up.reshape(C, KC, CH).transpose(1, 0, 2)
    xflat = _x_chunks(node_feats, W_upr)
    wflat = _edge_weights(edge_feats, edge_attrs, W1, b1[None, :],
                          W2, b2[None, :], W3, b3[None, :],
                          W4f, b4f).reshape(KC * E2, 4 * CH)
    msgc = _sc_scatter(xflat, wflat, sh3, sender, recv)
    out4 = _post(msgc.reshape(KC, NP, 4 * CH), node_attrs,
                 WL0s, WL1s, WS0r, WS1r)
    return jnp.transpose(out4, (1, 2, 0))


# speedup vs baseline: 1.2083x; 1.1354x over previous
import functools

import jax
import jax.numpy as jnp
from jax import lax
from jax.experimental import pallas as pl
from jax.experimental.pallas import tpu as pltpu
from jax.experimental.pallas import tpu_sc as plsc

N = 10000
E = 160000
C = 128
A = 10
KC = 4
CH = C // KC
AVG = 16.0

NC = 2
NS = 16
EP = E // NS
BE = 80
NB = EP // BE
NP = 10240
NPT = NP // NS


def _xup_body(nf_ref, wup_ref, out_ref):
    out_ref[...] = jnp.dot(nf_ref[...], wup_ref[0],
                           preferred_element_type=jnp.float32)


def _x_chunks(node_feats, W_up):
    BN = 1000
    return pl.pallas_call(
        _xup_body,
        grid=(KC, N // BN),
        in_specs=[
            pl.BlockSpec((BN, C), lambda k, i: (i, 0)),
            pl.BlockSpec((1, C, CH), lambda k, i: (k, 0, 0)),
        ],
        out_specs=pl.BlockSpec((BN, CH), lambda k, i: (k * (N // BN) + i, 0)),
        out_shape=jax.ShapeDtypeStruct((KC * N, CH), jnp.float32),
    )(node_feats, W_up)


def _mlp_body(efa_ref, efb_ref, sa_ref, sb_ref, w1_ref, b1_ref, w2_ref,
              b2_ref, w3_ref, b3_ref, w4_ref, b4_ref, out_ref):
    big = jnp.concatenate([efa_ref[...], sa_ref[0:1], efb_ref[...],
                           sb_ref[0:1]], axis=0)
    tr = jnp.transpose(big)
    ef = jnp.concatenate([tr[:, 0:8], tr[:, 9:17]], axis=0)
    s0 = jnp.concatenate([tr[:, 8:9], tr[:, 17:18]], axis=0)
    h = jax.nn.silu(jnp.dot(ef, w1_ref[...],
                            preferred_element_type=jnp.float32) + b1_ref[...])
    h = jax.nn.silu(jnp.dot(h, w2_ref[...],
                            preferred_element_type=jnp.float32) + b2_ref[...])
    h = jax.nn.silu(jnp.dot(h, w3_ref[...],
                            preferred_element_type=jnp.float32) + b3_ref[...])
    w = jnp.dot(h, w4_ref[...], preferred_element_type=jnp.float32) \
        + b4_ref[...]
    col = lax.broadcasted_iota(jnp.int32, w.shape, 1)
    w = w * jnp.where((col % (2 * CH)) < CH, s0, 1.0)
    half = w.shape[0] // 2
    for k in range(KC):
        out_ref[k] = jnp.concatenate(
            [w[:half, k * 2 * CH:(k + 1) * 2 * CH],
             w[half:, k * 2 * CH:(k + 1) * 2 * CH]], axis=1)


def _edge_weights(edge_feats, edge_attrs, W1, b1, W2, b2, W3, b3, W4f, b4f):
    E2 = E // 2
    BEm = 3200
    nb = E2 // BEm
    return pl.pallas_call(
        _mlp_body,
        grid=(nb,),
        in_specs=[
            pl.BlockSpec((8, BEm), lambda i: (0, i)),
            pl.BlockSpec((8, BEm), lambda i, _nb=nb: (0, _nb + i)),
            pl.BlockSpec((4, BEm), lambda i: (0, i)),
            pl.BlockSpec((4, BEm), lambda i, _nb=nb: (0, _nb + i)),
            pl.BlockSpec((8, 64), lambda i: (0, 0)),
            pl.BlockSpec((1, 64), lambda i: (0, 0)),
            pl.BlockSpec((64, 64), lambda i: (0, 0)),
            pl.BlockSpec((1, 64), lambda i: (0, 0)),
            pl.BlockSpec((64, 64), lambda i: (0, 0)),
            pl.BlockSpec((1, 64), lambda i: (0, 0)),
            pl.BlockSpec((64, 2 * C), lambda i: (0, 0)),
            pl.BlockSpec((1, 2 * C), lambda i: (0, 0)),
        ],
        out_specs=pl.BlockSpec((KC, BEm, 4 * CH), lambda i: (0, i, 0)),
        out_shape=jax.ShapeDtypeStruct((KC, E2, 4 * CH), jnp.float32),
    )(jnp.transpose(edge_feats), jnp.transpose(edge_feats),
      jnp.transpose(edge_attrs), jnp.transpose(edge_attrs),
      W1, b1, W2, b2, W3, b3, W4f, b4f)


def _sc_body(xflat, wflat, sh3, send, recv, out,
             isv0, isv1, rcv0, rcv1, xj0, xj1, w0_, w1_, sh0_, sh1_,
             m0_, m1_, z_v, acc_sh,
             is0, is1, ls0, ls1, gs0, gs1, ss0, ss1):
    c = lax.axis_index("c")
    s = lax.axis_index("s")
    isv = (isv0, isv1)
    rcv = (rcv0, rcv1)
    xj = (xj0, xj1)
    w_v = (w0_, w1_)
    sh_v = (sh0_, sh1_)
    m_v = (m0_, m1_)
    is_ = (is0, is1)
    ls = (ls0, ls1)
    gs = (gs0, gs1)
    ss = (ss0, ss1)

    def _zb(i, _):
        r = i // (C // 16)
        g = i % (C // 16)
        z_v[r, pl.ds(g * 16, 16)] = jnp.zeros((16,), jnp.float32)
        return 0
    lax.fori_loop(0, 32 * (C // 16), _zb, 0)

    for kk in range(KC // NC):
        chunk = NC * c + kk
        for t in range(NPT // 32):
            pltpu.sync_copy(z_v, acc_sh.at[pl.ds(s * NPT + t * 32, 32)])
        plsc.subcore_barrier()

        xoff = chunk * N
        woff = chunk * (E // 2)
        ebase = s * EP

        H = BE // 2

        def is_issue(b, q):
            bh = s * (EP // 2) + b * H
            pltpu.async_copy(send.at[pl.ds(bh, H)],
                             isv[q].at[pl.ds(0, H)], is_[q])
            pltpu.async_copy(send.at[pl.ds(E // 2 + bh, H)],
                             isv[q].at[pl.ds(H, H)], is_[q])

        def is_wait(q):
            pltpu.make_async_copy(send.at[pl.ds(0, H)],
                                  isv[q].at[pl.ds(0, H)], is_[q]).wait()
            pltpu.make_async_copy(send.at[pl.ds(0, H)],
                                  isv[q].at[pl.ds(H, H)], is_[q]).wait()

        def lin_issue(b, q):
            bh = s * (EP // 2) + b * H
            pltpu.async_copy(recv.at[pl.ds(bh, H)],
                             rcv[q].at[pl.ds(0, H)], ls[q])
            pltpu.async_copy(recv.at[pl.ds(E // 2 + bh, H)],
                             rcv[q].at[pl.ds(H, H)], ls[q])
            pltpu.async_copy(sh3.at[pl.ds(3 * bh, 3 * H)],
                             sh_v[q].at[pl.ds(0, 3 * H)], ls[q])
            pltpu.async_copy(sh3.at[pl.ds(3 * (E // 2) + 3 * bh, 3 * H)],
                             sh_v[q].at[pl.ds(3 * H, 3 * H)], ls[q])
            pltpu.async_copy(wflat.at[pl.ds(woff + bh, H)], w_v[q], ls[q])

        def lin_wait(q):
            pltpu.make_async_copy(recv.at[pl.ds(0, H)],
                                  rcv[q].at[pl.ds(0, H)], ls[q]).wait()
            pltpu.make_async_copy(recv.at[pl.ds(0, H)],
                                  rcv[q].at[pl.ds(H, H)], ls[q]).wait()
            pltpu.make_async_copy(sh3.at[pl.ds(0, 3 * H)],
                                  sh_v[q].at[pl.ds(0, 3 * H)], ls[q]).wait()
            pltpu.make_async_copy(sh3.at[pl.ds(0, 3 * H)],
                                  sh_v[q].at[pl.ds(3 * H, 3 * H)],
                                  ls[q]).wait()
            pltpu.make_async_copy(wflat.at[pl.ds(0, H)], w_v[q],
                                  ls[q]).wait()

        def gather_issue(q):
            for gg in range(BE // 16):
                isv[q][pl.ds(gg * 16, 16)] = (
                    isv[q][pl.ds(gg * 16, 16)] + xoff)
            pltpu.async_copy(xflat.at[isv[q]], xj[q], gs[q])

        def gather_wait(q):
            pltpu.make_async_copy(xflat.at[isv[q]], xj[q], gs[q]).wait()

        def scatter_issue(q):
            pltpu.async_copy(m_v[q], acc_sh.at[rcv[q]], ss[q], add=True)

        def scatter_wait(q):
            pltpu.make_async_copy(m_v[q], acc_sh.at[rcv[q]], ss[q]).wait()

        def compute(q):
            xjq, wq, shq, mq = xj[q], w_v[q], sh_v[q], m_v[q]

            def _pair(ep, _):
                for half in range(2):
                    e = ep + half * (BE // 2)
                    coff = half * 2 * CH
                    soff = half * BE
                    shx = shq[pl.ds(e + soff, 16)][0]
                    shy = shq[pl.ds(e + soff + BE // 2, 16)][0]
                    shz = shq[pl.ds(e + soff + BE, 16)][0]
                    for j in range(CH // 16):
                        xv = xjq[e, pl.ds(j * 16, 16)]
                        w0v = wq[ep, pl.ds(coff + j * 16, 16)]
                        w1v = wq[ep, pl.ds(coff + CH + j * 16, 16)]
                        mq[e, pl.ds(j * 16, 16)] = w0v * xv
                        t1 = w1v * xv
                        mq[e, pl.ds(CH + j * 16, 16)] = t1 * shx
                        mq[e, pl.ds(2 * CH + j * 16, 16)] = t1 * shy
                        mq[e, pl.ds(3 * CH + j * 16, 16)] = t1 * shz
                return 0
            lax.fori_loop(0, BE // 2, _pair, 0)

        is_issue(0, 0)
        is_issue(1, 1)
        lin_issue(0, 0)
        is_wait(0)
        gather_issue(0)

        def pair(t, _):
            for p in (0, 1):
                g = 2 * t + p
                lin_wait(p)
                gather_wait(p)
                compute(p)
                scatter_issue(p)
                if p == 0:
                    @pl.when(g > 0)
                    def _():
                        scatter_wait(1)
                else:
                    scatter_wait(0)
                is_wait(1 - p)
                gather_issue(1 - p)
                lin_issue(g + 1, 1 - p)
                if p == 0:
                    is_issue(g + 2, p)
                else:
                    @pl.when(g < NB - 2)
                    def _():
                        is_issue(g + 2, p)
            return 0
        lax.fori_loop(0, NB // 2, pair, 0)
        lin_wait(0)
        gather_wait(0)
        compute(0)
        scatter_issue(0)
        scatter_wait(1)
        scatter_wait(0)

        plsc.subcore_barrier()
        pltpu.sync_copy(acc_sh.at[pl.ds(s * NPT, NPT)],
                        out.at[pl.ds(chunk * NP + s * NPT, NPT)])


def _sc_scatter(xflat, wflat, sh3, send, recv):
    mesh = plsc.VectorSubcoreMesh(core_axis_name="c", subcore_axis_name="s")
    run = functools.partial(
        pl.kernel,
        out_type=jax.ShapeDtypeStruct((KC * NP, 4 * CH), jnp.float32),
        mesh=mesh,
        scratch_types=(
            [pltpu.VMEM((BE,), jnp.int32)] * 4
            + [pltpu.VMEM((BE, CH), jnp.float32)] * 2
            + [pltpu.VMEM((BE // 2, 4 * CH), jnp.float32)] * 2
            + [pltpu.VMEM((3 * BE + 16,), jnp.float32)] * 2
            + [pltpu.VMEM((BE, 4 * CH), jnp.float32)] * 2
            + [pltpu.VMEM((32, 4 * CH), jnp.float32)]
            + [pltpu.VMEM_SHARED((NP, 4 * CH), jnp.float32)]
            + [pltpu.SemaphoreType.DMA] * 8
        ),
        compiler_params=pltpu.CompilerParams(use_tc_tiling_on_sc=False),
    )(_sc_body)
    return run(xflat, wflat, sh3, send, recv)


def _post_body(msg_ref, attrs_ref, wl0_ref, wl1_ref, ws0_ref, ws1_ref, out_ref):
    attrs = attrs_ref[...]

    def skip(m, ws_ref):
        t = jnp.concatenate(
            [(m * attrs[:, v:v + 1]).astype(jnp.bfloat16) for v in range(A)],
            axis=1)
        return jnp.dot(t, ws_ref[...], preferred_element_type=jnp.float32)

    for i in range(4):
        m = jnp.concatenate(
            [msg_ref[k][:, i * CH:(i + 1) * CH] for k in range(KC)], axis=1)
        wl = wl0_ref if i == 0 else wl1_ref
        m = jnp.dot(m, wl[...], preferred_element_type=jnp.float32)
        out_ref[i] = skip(m, ws0_ref if i == 0 else ws1_ref)


def _post(msgc, node_attrs, WL0s, WL1s, WS0r, WS1r):
    BN = 400
    return pl.pallas_call(
        _post_body,
        grid=(N // BN,),
        in_specs=[
            pl.BlockSpec((KC, BN, 4 * CH), lambda i: (0, i, 0)),
            pl.BlockSpec((BN, A), lambda i: (i, 0)),
            pl.BlockSpec((C, C), lambda i: (0, 0)),
            pl.BlockSpec((C, C), lambda i: (0, 0)),
            pl.BlockSpec((A * C, C), lambda i: (0, 0)),
            pl.BlockSpec((A * C, C), lambda i: (0, 0)),
        ],
        out_specs=pl.BlockSpec((4, BN, C), lambda i: (0, i, 0)),
        out_shape=jax.ShapeDtypeStruct((4, N, C), jnp.float32),
    )(msgc, node_attrs, WL0s, WL1s, WS0r, WS1r)


def kernel(node_attrs, node_feats, edge_attrs, edge_feats, edge_index,
           W_up, W1, b1, W2, b2, W3, b3, W4, b4, WL0, WL1, WS0, WS1):
    cols = jnp.concatenate(
        [jnp.concatenate([jnp.arange(k * CH, (k + 1) * CH),
                          jnp.arange(C + k * CH, C + (k + 1) * CH)])
         for k in range(KC)])
    W4f = W4[:, cols]
    b4f = b4[cols][None, :]
    WL0s = WL0 / AVG
    WL1s = WL1 / AVG
    WS0r = jnp.transpose(WS0, (1, 0, 2)).reshape(A * C, C).astype(jnp.bfloat16)
    WS1r = jnp.transpose(WS1, (1, 0, 2)).reshape(A * C, C).astype(jnp.bfloat16)

    E2 = E // 2
    sender = edge_index[0].astype(jnp.int32)
    recv = edge_index[1].astype(jnp.int32)
    H = BE // 2
    sh3 = jnp.transpose(edge_attrs[:, 1:4].reshape(E // H, H, 3),
                        (0, 2, 1)).reshape(3 * E)

    W_upr = W_up.reshape(C, KC, CH).transpose(1, 0, 2)
    xflat = _x_chunks(node_feats, W_upr)
    wflat = _edge_weights(edge_feats, edge_attrs, W1, b1[None, :],
                          W2, b2[None, :], W3, b3[None, :],
                          W4f, b4f).reshape(KC * E2, 4 * CH)
    msgc = _sc_scatter(xflat, wflat, sh3, sender, recv)
    out4 = _post(msgc.reshape(KC, NP, 4 * CH), node_attrs,
                 WL0s, WL1s, WS0r, WS1r)
    return jnp.transpose(out4, (1, 2, 0))


# speedup vs baseline: 1.2214x; 1.0108x over previous
import functools

import jax
import jax.numpy as jnp
from jax import lax
from jax.experimental import pallas as pl
from jax.experimental.pallas import tpu as pltpu
from jax.experimental.pallas import tpu_sc as plsc

N = 10000
E = 160000
C = 128
A = 10
KC = 4
CH = C // KC
AVG = 16.0

NC = 2
NS = 16
EP = E // NS
BE = 80
NB = EP // BE
NP = 10240
NPT = NP // NS


def _xup_body(nf_ref, wup_ref, out_ref):
    out_ref[...] = jnp.dot(nf_ref[...], wup_ref[0],
                           preferred_element_type=jnp.float32)


def _x_chunks(node_feats, W_up):
    BN = 1000
    return pl.pallas_call(
        _xup_body,
        grid=(KC, N // BN),
        in_specs=[
            pl.BlockSpec((BN, C), lambda k, i: (i, 0)),
            pl.BlockSpec((1, C, CH), lambda k, i: (k, 0, 0)),
        ],
        out_specs=pl.BlockSpec((BN, CH), lambda k, i: (k * (N // BN) + i, 0)),
        out_shape=jax.ShapeDtypeStruct((KC * N, CH), jnp.float32),
    )(node_feats, W_up)


def _mlp_body(efa_ref, efb_ref, sa_ref, sb_ref, w1_ref, b1_ref, w2_ref,
              b2_ref, w3_ref, b3_ref, w4_ref, b4_ref, out_ref):
    big = jnp.concatenate([efa_ref[...], sa_ref[0:1], efb_ref[...],
                           sb_ref[0:1]], axis=0)
    tr = jnp.transpose(big)
    ef = jnp.concatenate([tr[:, 0:8], tr[:, 9:17]], axis=0)
    s0 = jnp.concatenate([tr[:, 8:9], tr[:, 17:18]], axis=0)
    h = jax.nn.silu(jnp.dot(ef, w1_ref[...],
                            preferred_element_type=jnp.float32) + b1_ref[...])
    h = jax.nn.silu(jnp.dot(h, w2_ref[...],
                            preferred_element_type=jnp.float32) + b2_ref[...])
    h = jax.nn.silu(jnp.dot(h, w3_ref[...],
                            preferred_element_type=jnp.float32) + b3_ref[...])
    w = jnp.dot(h, w4_ref[...], preferred_element_type=jnp.float32) \
        + b4_ref[...]
    col = lax.broadcasted_iota(jnp.int32, w.shape, 1)
    w = w * jnp.where((col % (2 * CH)) < CH, s0, 1.0)
    half = w.shape[0] // 2
    for k in range(KC):
        out_ref[k] = jnp.concatenate(
            [w[:half, k * 2 * CH:(k + 1) * 2 * CH],
             w[half:, k * 2 * CH:(k + 1) * 2 * CH]], axis=1)


def _edge_weights(edge_feats, edge_attrs, W1, b1, W2, b2, W3, b3, W4f, b4f):
    E2 = E // 2
    BEm = 3200
    nb = E2 // BEm
    return pl.pallas_call(
        _mlp_body,
        grid=(nb,),
        in_specs=[
            pl.BlockSpec((8, BEm), lambda i: (0, i)),
            pl.BlockSpec((8, BEm), lambda i, _nb=nb: (0, _nb + i)),
            pl.BlockSpec((4, BEm), lambda i: (0, i)),
            pl.BlockSpec((4, BEm), lambda i, _nb=nb: (0, _nb + i)),
            pl.BlockSpec((8, 64), lambda i: (0, 0)),
            pl.BlockSpec((1, 64), lambda i: (0, 0)),
            pl.BlockSpec((64, 64), lambda i: (0, 0)),
            pl.BlockSpec((1, 64), lambda i: (0, 0)),
            pl.BlockSpec((64, 64), lambda i: (0, 0)),
            pl.BlockSpec((1, 64), lambda i: (0, 0)),
            pl.BlockSpec((64, 2 * C), lambda i: (0, 0)),
            pl.BlockSpec((1, 2 * C), lambda i: (0, 0)),
        ],
        out_specs=pl.BlockSpec((KC, BEm, 4 * CH), lambda i: (0, i, 0)),
        out_shape=jax.ShapeDtypeStruct((KC, E2, 4 * CH), jnp.float32),
    )(jnp.transpose(edge_feats), jnp.transpose(edge_feats),
      jnp.transpose(edge_attrs), jnp.transpose(edge_attrs),
      W1, b1, W2, b2, W3, b3, W4f, b4f)


def _sc_body(xflat, wflat, sh3, send, recv, out,
             isv0, isv1, rcv0, rcv1, xj0, xj1, w0_, w1_, sh0_, sh1_,
             m0_, m1_, z_v, acc_sh,
             is0, is1, ls0, ls1, gs0, gs1, ss0, ss1):
    c = lax.axis_index("c")
    s = lax.axis_index("s")
    isv = (isv0, isv1)
    rcv = (rcv0, rcv1)
    xj = (xj0, xj1)
    w_v = (w0_, w1_)
    sh_v = (sh0_, sh1_)
    m_v = (m0_, m1_)
    is_ = (is0, is1)
    ls = (ls0, ls1)
    gs = (gs0, gs1)
    ss = (ss0, ss1)

    def _zb(i, _):
        r = i // (C // 16)
        g = i % (C // 16)
        z_v[r, pl.ds(g * 16, 16)] = jnp.zeros((16,), jnp.float32)
        return 0
    lax.fori_loop(0, 32 * (C // 16), _zb, 0)

    for kk in range(KC // NC):
        chunk = NC * c + kk
        for t in range(NPT // 32):
            pltpu.sync_copy(z_v, acc_sh.at[pl.ds(s * NPT + t * 32, 32)])
        plsc.subcore_barrier()

        xoff = chunk * N
        woff = chunk * (E // 2)
        ebase = s * EP

        H = BE // 2

        def is_issue(b, q):
            bh = s * (EP // 2) + b * H
            pltpu.async_copy(send.at[pl.ds(bh, H)],
                             isv[q].at[pl.ds(0, H)], is_[q])
            pltpu.async_copy(send.at[pl.ds(E // 2 + bh, H)],
                             isv[q].at[pl.ds(H, H)], is_[q])

        def is_wait(q):
            pltpu.make_async_copy(send.at[pl.ds(0, H)],
                                  isv[q].at[pl.ds(0, H)], is_[q]).wait()
            pltpu.make_async_copy(send.at[pl.ds(0, H)],
                                  isv[q].at[pl.ds(H, H)], is_[q]).wait()

        def lin_issue(b, q):
            bh = s * (EP // 2) + b * H
            pltpu.async_copy(recv.at[pl.ds(bh, H)],
                             rcv[q].at[pl.ds(0, H)], ls[q])
            pltpu.async_copy(recv.at[pl.ds(E // 2 + bh, H)],
                             rcv[q].at[pl.ds(H, H)], ls[q])
            pltpu.async_copy(sh3.at[pl.ds(3 * bh, 3 * H)],
                             sh_v[q].at[pl.ds(0, 3 * H)], ls[q])
            pltpu.async_copy(sh3.at[pl.ds(3 * (E // 2) + 3 * bh, 3 * H)],
                             sh_v[q].at[pl.ds(3 * H, 3 * H)], ls[q])
            pltpu.async_copy(wflat.at[pl.ds(woff + bh, H)], w_v[q], ls[q])

        def lin_wait(q):
            pltpu.make_async_copy(recv.at[pl.ds(0, H)],
                                  rcv[q].at[pl.ds(0, H)], ls[q]).wait()
            pltpu.make_async_copy(recv.at[pl.ds(0, H)],
                                  rcv[q].at[pl.ds(H, H)], ls[q]).wait()
            pltpu.make_async_copy(sh3.at[pl.ds(0, 3 * H)],
                                  sh_v[q].at[pl.ds(0, 3 * H)], ls[q]).wait()
            pltpu.make_async_copy(sh3.at[pl.ds(0, 3 * H)],
                                  sh_v[q].at[pl.ds(3 * H, 3 * H)],
                                  ls[q]).wait()
            pltpu.make_async_copy(wflat.at[pl.ds(0, H)], w_v[q],
                                  ls[q]).wait()

        def gather_issue(q):
            for gg in range(BE // 16):
                isv[q][pl.ds(gg * 16, 16)] = (
                    isv[q][pl.ds(gg * 16, 16)] + xoff)
            pltpu.async_copy(xflat.at[isv[q]], xj[q], gs[q])

        def gather_wait(q):
            pltpu.make_async_copy(xflat.at[isv[q]], xj[q], gs[q]).wait()

        def scatter_issue(q):
            pltpu.async_copy(m_v[q], acc_sh.at[rcv[q]], ss[q], add=True)

        def scatter_wait(q):
            pltpu.make_async_copy(m_v[q], acc_sh.at[rcv[q]], ss[q]).wait()

        def compute(q):
            xjq, wq, shq, mq = xj[q], w_v[q], sh_v[q], m_v[q]

            def _pair(ep, _):
                for half in range(2):
                    e = ep + half * (BE // 2)
                    coff = half * 2 * CH
                    soff = half * BE
                    shx = shq[pl.ds(e + soff, 16)][0]
                    shy = shq[pl.ds(e + soff + BE // 2, 16)][0]
                    shz = shq[pl.ds(e + soff + BE, 16)][0]
                    for j in range(CH // 16):
                        xv = xjq[e, pl.ds(j * 16, 16)]
                        w0v = wq[ep, pl.ds(coff + j * 16, 16)]
                        w1v = wq[ep, pl.ds(coff + CH + j * 16, 16)]
                        mq[e, pl.ds(j * 16, 16)] = w0v * xv
                        t1 = w1v * xv
                        mq[e, pl.ds(CH + j * 16, 16)] = t1 * shx
                        mq[e, pl.ds(2 * CH + j * 16, 16)] = t1 * shy
                        mq[e, pl.ds(3 * CH + j * 16, 16)] = t1 * shz
                return 0
            lax.fori_loop(0, BE // 2, _pair, 0)

        is_issue(0, 0)
        is_issue(1, 1)
        lin_issue(0, 0)
        is_wait(0)
        gather_issue(0)

        def pair(t, _):
            for p in (0, 1):
                g = 2 * t + p
                lin_wait(p)
                gather_wait(p)
                is_wait(1 - p)
                gather_issue(1 - p)
                compute(p)
                scatter_issue(p)
                if p == 0:
                    @pl.when(g > 0)
                    def _():
                        scatter_wait(1)
                else:
                    scatter_wait(0)
                lin_issue(g + 1, 1 - p)
                if p == 0:
                    is_issue(g + 2, p)
                else:
                    @pl.when(g < NB - 2)
                    def _():
                        is_issue(g + 2, p)
            return 0
        lax.fori_loop(0, NB // 2, pair, 0)
        lin_wait(0)
        gather_wait(0)
        compute(0)
        scatter_issue(0)
        scatter_wait(1)
        scatter_wait(0)

        plsc.subcore_barrier()
        pltpu.sync_copy(acc_sh.at[pl.ds(s * NPT, NPT)],
                        out.at[pl.ds(chunk * NP + s * NPT, NPT)])


def _sc_scatter(xflat, wflat, sh3, send, recv):
    mesh = plsc.VectorSubcoreMesh(core_axis_name="c", subcore_axis_name="s")
    run = functools.partial(
        pl.kernel,
        out_type=jax.ShapeDtypeStruct((KC * NP, 4 * CH), jnp.float32),
        mesh=mesh,
        scratch_types=(
            [pltpu.VMEM((BE,), jnp.int32)] * 4
            + [pltpu.VMEM((BE, CH), jnp.float32)] * 2
            + [pltpu.VMEM((BE // 2, 4 * CH), jnp.float32)] * 2
            + [pltpu.VMEM((3 * BE + 16,), jnp.float32)] * 2
            + [pltpu.VMEM((BE, 4 * CH), jnp.float32)] * 2
            + [pltpu.VMEM((32, 4 * CH), jnp.float32)]
            + [pltpu.VMEM_SHARED((NP, 4 * CH), jnp.float32)]
            + [pltpu.SemaphoreType.DMA] * 8
        ),
        compiler_params=pltpu.CompilerParams(use_tc_tiling_on_sc=False),
    )(_sc_body)
    return run(xflat, wflat, sh3, send, recv)


def _post_body(msg_ref, attrs_ref, wl0_ref, wl1_ref, ws0_ref, ws1_ref, out_ref):
    attrs = attrs_ref[...]

    def skip(m, ws_ref):
        t = jnp.concatenate(
            [(m * attrs[:, v:v + 1]).astype(jnp.bfloat16) for v in range(A)],
            axis=1)
        return jnp.dot(t, ws_ref[...], preferred_element_type=jnp.float32)

    for i in range(4):
        m = jnp.concatenate(
            [msg_ref[k][:, i * CH:(i + 1) * CH] for k in range(KC)], axis=1)
        wl = wl0_ref if i == 0 else wl1_ref
        m = jnp.dot(m, wl[...], preferred_element_type=jnp.float32)
        out_ref[i] = skip(m, ws0_ref if i == 0 else ws1_ref)


def _post(msgc, node_attrs, WL0s, WL1s, WS0r, WS1r):
    BN = 400
    return pl.pallas_call(
        _post_body,
        grid=(N // BN,),
        in_specs=[
            pl.BlockSpec((KC, BN, 4 * CH), lambda i: (0, i, 0)),
            pl.BlockSpec((BN, A), lambda i: (i, 0)),
            pl.BlockSpec((C, C), lambda i: (0, 0)),
            pl.BlockSpec((C, C), lambda i: (0, 0)),
            pl.BlockSpec((A * C, C), lambda i: (0, 0)),
            pl.BlockSpec((A * C, C), lambda i: (0, 0)),
        ],
        out_specs=pl.BlockSpec((4, BN, C), lambda i: (0, i, 0)),
        out_shape=jax.ShapeDtypeStruct((4, N, C), jnp.float32),
    )(msgc, node_attrs, WL0s, WL1s, WS0r, WS1r)


def kernel(node_attrs, node_feats, edge_attrs, edge_feats, edge_index,
           W_up, W1, b1, W2, b2, W3, b3, W4, b4, WL0, WL1, WS0, WS1):
    cols = jnp.concatenate(
        [jnp.concatenate([jnp.arange(k * CH, (k + 1) * CH),
                          jnp.arange(C + k * CH, C + (k + 1) * CH)])
         for k in range(KC)])
    W4f = W4[:, cols]
    b4f = b4[cols][None, :]
    WL0s = WL0 / AVG
    WL1s = WL1 / AVG
    WS0r = jnp.transpose(WS0, (1, 0, 2)).reshape(A * C, C).astype(jnp.bfloat16)
    WS1r = jnp.transpose(WS1, (1, 0, 2)).reshape(A * C, C).astype(jnp.bfloat16)

    E2 = E // 2
    sender = edge_index[0].astype(jnp.int32)
    recv = edge_index[1].astype(jnp.int32)
    H = BE // 2
    sh3 = jnp.transpose(edge_attrs[:, 1:4].reshape(E // H, H, 3),
                        (0, 2, 1)).reshape(3 * E)

    W_upr = W_up.reshape(C, KC, CH).transpose(1, 0, 2)
    xflat = _x_chunks(node_feats, W_upr)
    wflat = _edge_weights(edge_feats, edge_attrs, W1, b1[None, :],
                          W2, b2[None, :], W3, b3[None, :],
                          W4f, b4f).reshape(KC * E2, 4 * CH)
    msgc = _sc_scatter(xflat, wflat, sh3, sender, recv)
    out4 = _post(msgc.reshape(KC, NP, 4 * CH), node_attrs,
                 WL0s, WL1s, WS0r, WS1r)
    return jnp.transpose(out4, (1, 2, 0))


# speedup vs baseline: 1.2782x; 1.0466x over previous
import functools

import jax
import jax.numpy as jnp
from jax import lax
from jax.experimental import pallas as pl
from jax.experimental.pallas import tpu as pltpu
from jax.experimental.pallas import tpu_sc as plsc

N = 10000
E = 160000
C = 128
A = 10
KC = 4
CH = C // KC
AVG = 16.0

NC = 2
NS = 16
EP = E // NS
BE = 80
NB = EP // BE
NP = 10240
NPT = NP // NS


def _xup_body(nf_ref, wup_ref, out_ref):
    out_ref[...] = jnp.dot(nf_ref[...], wup_ref[0],
                           preferred_element_type=jnp.float32)


def _x_chunks(node_feats, W_up):
    BN = 1000
    return pl.pallas_call(
        _xup_body,
        grid=(KC, N // BN),
        in_specs=[
            pl.BlockSpec((BN, C), lambda k, i: (i, 0)),
            pl.BlockSpec((1, C, CH), lambda k, i: (k, 0, 0)),
        ],
        out_specs=pl.BlockSpec((BN, CH), lambda k, i: (k * (N // BN) + i, 0)),
        out_shape=jax.ShapeDtypeStruct((KC * N, CH), jnp.float32),
    )(node_feats, W_up)


def _mlp_body(efa_ref, efb_ref, sa_ref, sb_ref, w1_ref, b1_ref, w2_ref,
              b2_ref, w3_ref, b3_ref, w4_ref, b4_ref, out_ref):
    big = jnp.concatenate([efa_ref[...], sa_ref[0:1], efb_ref[...],
                           sb_ref[0:1]], axis=0)
    tr = jnp.transpose(big)
    ef = jnp.concatenate([tr[:, 0:8], tr[:, 9:17]], axis=0)
    s0 = jnp.concatenate([tr[:, 8:9], tr[:, 17:18]], axis=0)
    h = jax.nn.silu(jnp.dot(ef, w1_ref[...],
                            preferred_element_type=jnp.float32) + b1_ref[...])
    h = jax.nn.silu(jnp.dot(h, w2_ref[...],
                            preferred_element_type=jnp.float32) + b2_ref[...])
    h = jax.nn.silu(jnp.dot(h, w3_ref[...],
                            preferred_element_type=jnp.float32) + b3_ref[...])
    w = jnp.dot(h, w4_ref[...], preferred_element_type=jnp.float32) \
        + b4_ref[...]
    col = lax.broadcasted_iota(jnp.int32, w.shape, 1)
    w = w * jnp.where((col % (2 * CH)) < CH, s0, 1.0)
    half = w.shape[0] // 2
    for k in range(KC):
        out_ref[k] = jnp.concatenate(
            [w[:half, k * 2 * CH:(k + 1) * 2 * CH],
             w[half:, k * 2 * CH:(k + 1) * 2 * CH]], axis=1)


def _edge_weights(edge_feats, edge_attrs, W1, b1, W2, b2, W3, b3, W4f, b4f):
    E2 = E // 2
    BEm = 3200
    nb = E2 // BEm
    return pl.pallas_call(
        _mlp_body,
        grid=(nb,),
        in_specs=[
            pl.BlockSpec((8, BEm), lambda i: (0, i)),
            pl.BlockSpec((8, BEm), lambda i, _nb=nb: (0, _nb + i)),
            pl.BlockSpec((4, BEm), lambda i: (0, i)),
            pl.BlockSpec((4, BEm), lambda i, _nb=nb: (0, _nb + i)),
            pl.BlockSpec((8, 64), lambda i: (0, 0)),
            pl.BlockSpec((1, 64), lambda i: (0, 0)),
            pl.BlockSpec((64, 64), lambda i: (0, 0)),
            pl.BlockSpec((1, 64), lambda i: (0, 0)),
            pl.BlockSpec((64, 64), lambda i: (0, 0)),
            pl.BlockSpec((1, 64), lambda i: (0, 0)),
            pl.BlockSpec((64, 2 * C), lambda i: (0, 0)),
            pl.BlockSpec((1, 2 * C), lambda i: (0, 0)),
        ],
        out_specs=pl.BlockSpec((KC, BEm, 4 * CH), lambda i: (0, i, 0)),
        out_shape=jax.ShapeDtypeStruct((KC, E2, 4 * CH), jnp.float32),
    )(jnp.transpose(edge_feats), jnp.transpose(edge_feats),
      jnp.transpose(edge_attrs), jnp.transpose(edge_attrs),
      W1, b1, W2, b2, W3, b3, W4f, b4f)


def _sc_body(xflat, wflat, sh3, send, recv, out,
             isv0, isv1, rcv0, rcv1, xj0, xj1, w0_, w1_, sh0_, sh1_,
             m0_, m1_, z_v, acc_sh,
             is0, is1, ls0, ls1, gs0, gs1, ss0, ss1):
    c = lax.axis_index("c")
    s = lax.axis_index("s")
    isv = (isv0, isv1)
    rcv = (rcv0, rcv1)
    xj = (xj0, xj1)
    w_v = (w0_, w1_)
    sh_v = (sh0_, sh1_)
    m_v = (m0_, m1_)
    is_ = (is0, is1)
    ls = (ls0, ls1)
    gs = (gs0, gs1)
    ss = (ss0, ss1)

    def _zb(i, _):
        r = i // (C // 16)
        g = i % (C // 16)
        z_v[r, pl.ds(g * 16, 16)] = jnp.zeros((16,), jnp.float32)
        return 0
    lax.fori_loop(0, 32 * (C // 16), _zb, 0)

    for kk in range(KC // NC):
        chunk = NC * c + kk
        for t in range(NPT // 32):
            pltpu.sync_copy(z_v, acc_sh.at[pl.ds(s * NPT + t * 32, 32)])
        plsc.subcore_barrier()

        xoff = chunk * N
        woff = chunk * (E // 2)
        ebase = s * EP

        H = BE // 2

        def is_issue(b, q):
            bh = s * (EP // 2) + b * H
            pltpu.async_copy(send.at[pl.ds(bh, H)],
                             isv[q].at[pl.ds(0, H)], is_[q])
            pltpu.async_copy(send.at[pl.ds(E // 2 + bh, H)],
                             isv[q].at[pl.ds(H, H)], is_[q])

        def is_wait(q):
            pltpu.make_async_copy(send.at[pl.ds(0, H)],
                                  isv[q].at[pl.ds(0, H)], is_[q]).wait()
            pltpu.make_async_copy(send.at[pl.ds(0, H)],
                                  isv[q].at[pl.ds(H, H)], is_[q]).wait()

        def lin_issue(b, q):
            bh = s * (EP // 2) + b * H
            pltpu.async_copy(recv.at[pl.ds(bh, H)],
                             rcv[q].at[pl.ds(0, H)], ls[q])
            pltpu.async_copy(recv.at[pl.ds(E // 2 + bh, H)],
                             rcv[q].at[pl.ds(H, H)], ls[q])
            pltpu.async_copy(sh3.at[pl.ds(3 * bh, 3 * H)],
                             sh_v[q].at[pl.ds(0, 3 * H)], ls[q])
            pltpu.async_copy(sh3.at[pl.ds(3 * (E // 2) + 3 * bh, 3 * H)],
                             sh_v[q].at[pl.ds(3 * H, 3 * H)], ls[q])
            pltpu.async_copy(wflat.at[pl.ds(woff + bh, H)], w_v[q], ls[q])

        def lin_wait(q):
            pltpu.make_async_copy(recv.at[pl.ds(0, H)],
                                  rcv[q].at[pl.ds(0, H)], ls[q]).wait()
            pltpu.make_async_copy(recv.at[pl.ds(0, H)],
                                  rcv[q].at[pl.ds(H, H)], ls[q]).wait()
            pltpu.make_async_copy(sh3.at[pl.ds(0, 3 * H)],
                                  sh_v[q].at[pl.ds(0, 3 * H)], ls[q]).wait()
            pltpu.make_async_copy(sh3.at[pl.ds(0, 3 * H)],
                                  sh_v[q].at[pl.ds(3 * H, 3 * H)],
                                  ls[q]).wait()
            pltpu.make_async_copy(wflat.at[pl.ds(0, H)], w_v[q],
                                  ls[q]).wait()

        def gather_issue(q):
            for gg in range(BE // 16):
                isv[q][pl.ds(gg * 16, 16)] = (
                    isv[q][pl.ds(gg * 16, 16)] + xoff)
            pltpu.async_copy(xflat.at[isv[q]], xj[q], gs[q])

        def gather_wait(q):
            pltpu.make_async_copy(xflat.at[isv[q]], xj[q], gs[q]).wait()

        def scatter_issue(q):
            pltpu.async_copy(m_v[q], acc_sh.at[rcv[q]], ss[q], add=True)

        def scatter_wait(q):
            pltpu.make_async_copy(m_v[q], acc_sh.at[rcv[q]], ss[q]).wait()

        def compute(q):
            xjq, wq, shq, mq = xj[q], w_v[q], sh_v[q], m_v[q]

            def _pair(pp, _):
                for u in range(2):
                    ep = 2 * pp + u
                    for half in range(2):
                        e = ep + half * (BE // 2)
                        coff = half * 2 * CH
                        soff = half * BE
                        shx = shq[pl.ds(e + soff, 16)][0]
                        shy = shq[pl.ds(e + soff + BE // 2, 16)][0]
                        shz = shq[pl.ds(e + soff + BE, 16)][0]
                        for j in range(CH // 16):
                            xv = xjq[e, pl.ds(j * 16, 16)]
                            w0v = wq[ep, pl.ds(coff + j * 16, 16)]
                            w1v = wq[ep, pl.ds(coff + CH + j * 16, 16)]
                            mq[e, pl.ds(j * 16, 16)] = w0v * xv
                            t1 = w1v * xv
                            mq[e, pl.ds(CH + j * 16, 16)] = t1 * shx
                            mq[e, pl.ds(2 * CH + j * 16, 16)] = t1 * shy
                            mq[e, pl.ds(3 * CH + j * 16, 16)] = t1 * shz
                return 0
            lax.fori_loop(0, BE // 4, _pair, 0)

        is_issue(0, 0)
        is_issue(1, 1)
        lin_issue(0, 0)
        is_wait(0)
        gather_issue(0)

        def pair(t, _):
            for p in (0, 1):
                g = 2 * t + p
                lin_wait(p)
                gather_wait(p)
                is_wait(1 - p)
                gather_issue(1 - p)
                compute(p)
                scatter_issue(p)
                if p == 0:
                    @pl.when(g > 0)
                    def _():
                        scatter_wait(1)
                else:
                    scatter_wait(0)
                lin_issue(g + 1, 1 - p)
                if p == 0:
                    is_issue(g + 2, p)
                else:
                    @pl.when(g < NB - 2)
                    def _():
                        is_issue(g + 2, p)
            return 0
        lax.fori_loop(0, NB // 2, pair, 0)
        lin_wait(0)
        gather_wait(0)
        compute(0)
        scatter_issue(0)
        scatter_wait(1)
        scatter_wait(0)

        plsc.subcore_barrier()
        pltpu.sync_copy(acc_sh.at[pl.ds(s * NPT, NPT)],
                        out.at[pl.ds(chunk * NP + s * NPT, NPT)])


def _sc_scatter(xflat, wflat, sh3, send, recv):
    mesh = plsc.VectorSubcoreMesh(core_axis_name="c", subcore_axis_name="s")
    run = functools.partial(
        pl.kernel,
        out_type=jax.ShapeDtypeStruct((KC * NP, 4 * CH), jnp.float32),
        mesh=mesh,
        scratch_types=(
            [pltpu.VMEM((BE,), jnp.int32)] * 4
            + [pltpu.VMEM((BE, CH), jnp.float32)] * 2
            + [pltpu.VMEM((BE // 2, 4 * CH), jnp.float32)] * 2
            + [pltpu.VMEM((3 * BE + 16,), jnp.float32)] * 2
            + [pltpu.VMEM((BE, 4 * CH), jnp.float32)] * 2
            + [pltpu.VMEM((32, 4 * CH), jnp.float32)]
            + [pltpu.VMEM_SHARED((NP, 4 * CH), jnp.float32)]
            + [pltpu.SemaphoreType.DMA] * 8
        ),
        compiler_params=pltpu.CompilerParams(use_tc_tiling_on_sc=False),
    )(_sc_body)
    return run(xflat, wflat, sh3, send, recv)


def _post_body(msg_ref, attrs_ref, wl0_ref, wl1_ref, ws0_ref, ws1_ref, out_ref):
    attrs = attrs_ref[...]

    def skip(m, ws_ref):
        t = jnp.concatenate(
            [(m * attrs[:, v:v + 1]).astype(jnp.bfloat16) for v in range(A)],
            axis=1)
        return jnp.dot(t, ws_ref[...], preferred_element_type=jnp.float32)

    for i in range(4):
        m = jnp.concatenate(
            [msg_ref[k][:, i * CH:(i + 1) * CH] for k in range(KC)], axis=1)
        wl = wl0_ref if i == 0 else wl1_ref
        m = jnp.dot(m, wl[...], preferred_element_type=jnp.float32)
        out_ref[i] = skip(m, ws0_ref if i == 0 else ws1_ref)


def _post(msgc, node_attrs, WL0s, WL1s, WS0r, WS1r):
    BN = 400
    return pl.pallas_call(
        _post_body,
        grid=(N // BN,),
        in_specs=[
            pl.BlockSpec((KC, BN, 4 * CH), lambda i: (0, i, 0)),
            pl.BlockSpec((BN, A), lambda i: (i, 0)),
            pl.BlockSpec((C, C), lambda i: (0, 0)),
            pl.BlockSpec((C, C), lambda i: (0, 0)),
            pl.BlockSpec((A * C, C), lambda i: (0, 0)),
            pl.BlockSpec((A * C, C), lambda i: (0, 0)),
        ],
        out_specs=pl.BlockSpec((4, BN, C), lambda i: (0, i, 0)),
        out_shape=jax.ShapeDtypeStruct((4, N, C), jnp.float32),
    )(msgc, node_attrs, WL0s, WL1s, WS0r, WS1r)


def kernel(node_attrs, node_feats, edge_attrs, edge_feats, edge_index,
           W_up, W1, b1, W2, b2, W3, b3, W4, b4, WL0, WL1, WS0, WS1):
    cols = jnp.concatenate(
        [jnp.concatenate([jnp.arange(k * CH, (k + 1) * CH),
                          jnp.arange(C + k * CH, C + (k + 1) * CH)])
         for k in range(KC)])
    W4f = W4[:, cols]
    b4f = b4[cols][None, :]
    WL0s = WL0 / AVG
    WL1s = WL1 / AVG
    WS0r = jnp.transpose(WS0, (1, 0, 2)).reshape(A * C, C).astype(jnp.bfloat16)
    WS1r = jnp.transpose(WS1, (1, 0, 2)).reshape(A * C, C).astype(jnp.bfloat16)

    E2 = E // 2
    sender = edge_index[0].astype(jnp.int32)
    recv = edge_index[1].astype(jnp.int32)
    H = BE // 2
    sh3 = jnp.transpose(edge_attrs[:, 1:4].reshape(E // H, H, 3),
                        (0, 2, 1)).reshape(3 * E)

    W_upr = W_up.reshape(C, KC, CH).transpose(1, 0, 2)
    xflat = _x_chunks(node_feats, W_upr)
    wflat = _edge_weights(edge_feats, edge_attrs, W1, b1[None, :],
                          W2, b2[None, :], W3, b3[None, :],
                          W4f, b4f).reshape(KC * E2, 4 * CH)
    msgc = _sc_scatter(xflat, wflat, sh3, sender, recv)
    out4 = _post(msgc.reshape(KC, NP, 4 * CH), node_attrs,
                 WL0s, WL1s, WS0r, WS1r)
    return jnp.transpose(out4, (1, 2, 0))
